# Initial kernel scaffold; baseline (speedup 1.0000x reference)
import jax
import jax.numpy as jnp
from jax.experimental import pallas as pl


def kernel(embeddings, dis, target):
    raise NotImplementedError("write your pallas kernel here")


# speedup vs baseline: 1.9931x; 1.9931x over previous
import functools

import jax
import jax.numpy as jnp
from jax import lax
from jax.experimental import pallas as pl
from jax.experimental.pallas import tpu as pltpu

MARGIN = 0.2


def _block_body(R, NB, dis_ref, tcol_ref, trow_ref, emb_ref, embblk_ref, out_ref):
    i = pl.program_id(0)
    n = NB * R
    dis = dis_ref[...]
    tcol = tcol_ref[...]
    trow = trow_ref[...]
    col = lax.broadcasted_iota(jnp.int32, (R, n), 1)
    row = lax.broadcasted_iota(jnp.int32, (R, n), 0) + i * R
    same = tcol == trow
    neg_inf = jnp.float32(-jnp.inf)
    pos_inf = jnp.float32(jnp.inf)
    pval = jnp.where(same & (col != row), dis, neg_inf)
    nval = jnp.where(same, pos_inf, dis)
    pmax = jnp.max(pval, axis=1, keepdims=True)
    nmin = jnp.min(nval, axis=1, keepdims=True)
    big = jnp.int32(n)
    pidx = jnp.min(jnp.where(pval == pmax, col, big), axis=1, keepdims=True)
    nidx = jnp.min(jnp.where(nval == nmin, col, big), axis=1, keepdims=True)
    onehot_p = (col == pidx).astype(jnp.float32)
    onehot_n = (col == nidx).astype(jnp.float32)
    emb = emb_ref[...]
    p = jnp.dot(onehot_p, emb, preferred_element_type=jnp.float32)
    ng = jnp.dot(onehot_n, emb, preferred_element_type=jnp.float32)
    a = embblk_ref[...]
    ap = jnp.sum((a - p) ** 2, axis=1, keepdims=True)
    an = jnp.sum((a - ng) ** 2, axis=1, keepdims=True)
    losses = jnp.maximum(ap - an + jnp.float32(MARGIN), jnp.float32(0.0))
    part = jnp.sum(losses) / jnp.float32(n)

    @pl.when(i == 0)
    def _init():
        out_ref[0, 0] = jnp.float32(0.0)

    out_ref[0, 0] += part


def kernel(embeddings, dis, target):
    n, d = embeddings.shape
    R = 256
    NB = n // R
    tcol = target.reshape(n, 1)
    trow = target.reshape(1, n)
    out = pl.pallas_call(
        functools.partial(_block_body, R, NB),
        grid=(NB,),
        in_specs=[
            pl.BlockSpec((R, n), lambda i: (i, 0)),
            pl.BlockSpec((R, 1), lambda i: (i, 0)),
            pl.BlockSpec((1, n), lambda i: (0, 0)),
            pl.BlockSpec((n, d), lambda i: (0, 0)),
            pl.BlockSpec((R, d), lambda i: (i, 0)),
        ],
        out_specs=pl.BlockSpec(memory_space=pltpu.SMEM),
        out_shape=jax.ShapeDtypeStruct((1, 1), jnp.float32),
        compiler_params=pltpu.CompilerParams(
            dimension_semantics=("arbitrary",),
        ),
    )(dis, tcol, trow, embeddings, embeddings)
    return out[0, 0]
